# baseline (speedup 1.0000x reference)
import functools

import jax
import jax.numpy as jnp
from jax import lax
from jax.experimental import pallas as pl
from jax.experimental.pallas import tpu as pltpu
from jax.experimental.pallas import tpu_sc as plsc

N = 100
D = 131072
NW = 32
WSTRIPE = D // NW
CW = 4096
NCC = WSTRIPE // CW
L = 16
R = 8
UNITS = tuple((w, R) for w in range(0, 96, R)) + ((96, 4),)
NBUF = 3

_mesh = plsc.VectorSubcoreMesh(core_axis_name="c", subcore_axis_name="s")


@functools.partial(
    pl.kernel,
    out_type=jax.ShapeDtypeStruct((N, D), jnp.float32),
    mesh=_mesh,
    compiler_params=pltpu.CompilerParams(needs_layout_passes=False),
    scratch_types=[
        pltpu.VMEM((128,), jnp.int32),
        pltpu.VMEM((12, R), jnp.int32),
        pltpu.VMEM((4,), jnp.int32),
        [pltpu.VMEM((R, CW), jnp.float32) for _ in range(NBUF)],
        pltpu.VMEM((4, CW), jnp.float32),
        [pltpu.SemaphoreType.DMA for _ in range(NBUF)],
        [pltpu.SemaphoreType.DMA for _ in range(NBUF)],
    ],
)
def _sc_gather(x, ords, out, ordv, posv, tailposv, bufs, tailbuf,
               gsems, wsems):
    cid = lax.axis_index("c")
    sid = lax.axis_index("s")
    wid = sid * 2 + cid
    c0 = wid * WSTRIPE

    ordv[pl.ds(96, L)] = jnp.zeros((L,), jnp.int32)
    pltpu.sync_copy(ords, ordv.at[pl.ds(0, N)])

    iota = lax.iota(jnp.int32, L)
    for w in range(0, 112, L):
        iv = w + iota
        valid = iv < N
        ov = plsc.load_gather(ordv, [iv])
        main = jnp.logical_and(valid, ov < 96)
        plsc.store_scatter(posv, [lax.shift_right_logical(ov, 3),
                                  lax.bitwise_and(ov, R - 1)], iv, mask=main)
        tail = jnp.logical_and(valid, ov >= 96)
        plsc.store_scatter(tailposv, [ov - 96], iv, mask=tail)

    out_sl = out.at[:, pl.ds(c0, CW)]
    x_sl = x.at[:, pl.ds(c0, CW)]

    def _buf(s):
        w, r = UNITS[s]
        return bufs[s % NBUF] if r == R else tailbuf

    def _run(gather_src, scatter_dst):
        def start_read(s):
            pltpu.async_copy(gather_src(s), _buf(s), gsems[s % NBUF])

        scats = [None] * NBUF
        for s in range(min(NBUF - 1, len(UNITS))):
            start_read(s)
        for s in range(len(UNITS)):
            b = s % NBUF
            pltpu.make_async_copy(gather_src(s), _buf(s), gsems[b]).wait()
            scats[b] = pltpu.async_copy(_buf(s), scatter_dst(s), wsems[b])
            n = s + NBUF - 1
            if n < len(UNITS):
                nb = n % NBUF
                if scats[nb] is not None:
                    scats[nb].wait()
                    scats[nb] = None
                start_read(n)
        for b in range(NBUF):
            if scats[b] is not None:
                scats[b].wait()

    def inv_src(s):
        w, r = UNITS[s]
        return x.at[pl.ds(w, r), pl.ds(c0, CW)]

    def inv_dst(s):
        w, r = UNITS[s]
        return out_sl.at[posv.at[s]] if r == R else out_sl.at[tailposv]

    def fwd_src(s):
        w, r = UNITS[s]
        return x_sl.at[ordv.at[pl.ds(w, r)]]

    def fwd_dst(s):
        w, r = UNITS[s]
        return out.at[pl.ds(w, r), pl.ds(c0, CW)]

    @pl.when(wid < NW // 2)
    def _():
        _run(inv_src, inv_dst)

    @pl.when(wid >= NW // 2)
    def _():
        _run(fwd_src, fwd_dst)


def kernel(x, ordinals):
    return _sc_gather(x, ordinals)


# speedup vs baseline: 1.0266x; 1.0266x over previous
import functools

import jax
import jax.numpy as jnp
from jax import lax
from jax.experimental import pallas as pl
from jax.experimental.pallas import tpu as pltpu
from jax.experimental.pallas import tpu_sc as plsc

N = 100
D = 131072
NW = 32
WSTRIPE = D // NW
CW = 4096
NCC = WSTRIPE // CW
L = 16
R = 8
UNITS = tuple((w, R) for w in range(0, 96, R)) + ((96, 4),)
NBUF = 3

_mesh = plsc.VectorSubcoreMesh(core_axis_name="c", subcore_axis_name="s")


@functools.partial(
    pl.kernel,
    out_type=jax.ShapeDtypeStruct((N, D), jnp.float32),
    mesh=_mesh,
    compiler_params=pltpu.CompilerParams(needs_layout_passes=False),
    scratch_types=[
        pltpu.VMEM((128,), jnp.int32),
        pltpu.VMEM((12, R), jnp.int32),
        pltpu.VMEM((4,), jnp.int32),
        [pltpu.VMEM((R, CW), jnp.float32) for _ in range(NBUF)],
        pltpu.VMEM((4, CW), jnp.float32),
        [pltpu.SemaphoreType.DMA for _ in range(NBUF)],
        [pltpu.SemaphoreType.DMA for _ in range(NBUF)],
    ],
)
def _sc_gather(x, ords, out, ordv, posv, tailposv, bufs, tailbuf,
               gsems, wsems):
    cid = lax.axis_index("c")
    sid = lax.axis_index("s")
    wid = sid * 2 + cid
    c0 = wid * WSTRIPE

    ordv[pl.ds(96, L)] = jnp.zeros((L,), jnp.int32)
    pltpu.sync_copy(ords, ordv.at[pl.ds(0, N)])

    iota = lax.iota(jnp.int32, L)
    for w in range(0, 112, L):
        iv = w + iota
        valid = iv < N
        ov = plsc.load_gather(ordv, [iv])
        main = jnp.logical_and(valid, ov < 96)
        plsc.store_scatter(posv, [lax.shift_right_logical(ov, 3),
                                  lax.bitwise_and(ov, R - 1)], iv, mask=main)
        tail = jnp.logical_and(valid, ov >= 96)
        plsc.store_scatter(tailposv, [ov - 96], iv, mask=tail)

    out_sl = out.at[:, pl.ds(c0, CW)]

    def _buf(s):
        w, r = UNITS[s]
        return bufs[s % NBUF] if r == R else tailbuf

    def start_read(s):
        w, r = UNITS[s]
        pltpu.async_copy(x.at[pl.ds(w, r), pl.ds(c0, CW)],
                         _buf(s), gsems[s % NBUF])

    scats = [None] * NBUF
    for s in range(min(NBUF - 1, len(UNITS))):
        start_read(s)
    for s in range(len(UNITS)):
        b = s % NBUF
        w, r = UNITS[s]
        pltpu.make_async_copy(x.at[pl.ds(w, r), pl.ds(c0, CW)],
                              _buf(s), gsems[b]).wait()
        if r == R:
            dst = out_sl.at[posv.at[s]]
        else:
            dst = out_sl.at[tailposv]
        scats[b] = pltpu.async_copy(_buf(s), dst, wsems[b])
        n = s + NBUF - 1
        if n < len(UNITS):
            nb = n % NBUF
            if scats[nb] is not None:
                scats[nb].wait()
                scats[nb] = None
            start_read(n)
    for b in range(NBUF):
        if scats[b] is not None:
            scats[b].wait()


def kernel(x, ordinals):
    return _sc_gather(x, ordinals)


# speedup vs baseline: 1.0368x; 1.0100x over previous
import functools

import jax
import jax.numpy as jnp
from jax import lax
from jax.experimental import pallas as pl
from jax.experimental.pallas import tpu as pltpu
from jax.experimental.pallas import tpu_sc as plsc

N = 100
D = 131072
NW = 32
WSTRIPE = D // NW
CW = 4096
NCC = WSTRIPE // CW
L = 16
R = 8
UNITS = tuple((w, R) for w in range(0, 96, R)) + ((96, 4),)
NBUF = 3

_mesh = plsc.VectorSubcoreMesh(core_axis_name="c", subcore_axis_name="s")


@functools.partial(
    pl.kernel,
    out_type=jax.ShapeDtypeStruct((N, D), jnp.float32),
    mesh=_mesh,
    compiler_params=pltpu.CompilerParams(needs_layout_passes=False),
    scratch_types=[
        pltpu.VMEM((128,), jnp.int32),
        pltpu.VMEM((12, R), jnp.int32),
        pltpu.VMEM((4,), jnp.int32),
        [pltpu.VMEM((R, CW), jnp.float32) for _ in range(NBUF)],
        pltpu.VMEM((4, CW), jnp.float32),
        [pltpu.SemaphoreType.DMA for _ in range(NBUF)],
        [pltpu.SemaphoreType.DMA for _ in range(NBUF)],
    ],
)
def _sc_gather(x, ords, out, ordv, posv, tailposv, bufs, tailbuf,
               gsems, wsems):
    cid = lax.axis_index("c")
    sid = lax.axis_index("s")
    wid = sid * 2 + cid
    c0 = wid * WSTRIPE

    out_sl = out.at[:, pl.ds(c0, CW)]

    def _buf(s):
        w, r = UNITS[s]
        return bufs[s % NBUF] if r == R else tailbuf

    def start_read(s):
        w, r = UNITS[s]
        pltpu.async_copy(x.at[pl.ds(w, r), pl.ds(c0, CW)],
                         _buf(s), gsems[s % NBUF])

    scats = [None] * NBUF
    for s in range(min(NBUF - 1, len(UNITS))):
        start_read(s)

    ordv[pl.ds(96, L)] = jnp.zeros((L,), jnp.int32)
    pltpu.sync_copy(ords, ordv.at[pl.ds(0, N)])

    iota = lax.iota(jnp.int32, L)
    for w in range(0, 112, L):
        iv = w + iota
        valid = iv < N
        ov = plsc.load_gather(ordv, [iv])
        main = jnp.logical_and(valid, ov < 96)
        plsc.store_scatter(posv, [lax.shift_right_logical(ov, 3),
                                  lax.bitwise_and(ov, R - 1)], iv, mask=main)
        tail = jnp.logical_and(valid, ov >= 96)
        plsc.store_scatter(tailposv, [ov - 96], iv, mask=tail)
    for s in range(len(UNITS)):
        b = s % NBUF
        w, r = UNITS[s]
        pltpu.make_async_copy(x.at[pl.ds(w, r), pl.ds(c0, CW)],
                              _buf(s), gsems[b]).wait()
        if r == R:
            dst = out_sl.at[posv.at[s]]
        else:
            dst = out_sl.at[tailposv]
        scats[b] = pltpu.async_copy(_buf(s), dst, wsems[b])
        n = s + NBUF - 1
        if n < len(UNITS):
            nb = n % NBUF
            if scats[nb] is not None:
                scats[nb].wait()
                scats[nb] = None
            start_read(n)
    for b in range(NBUF):
        if scats[b] is not None:
            scats[b].wait()


def kernel(x, ordinals):
    return _sc_gather(x, ordinals)
